# baseline (speedup 1.0000x reference)
import functools

import jax
import jax.numpy as jnp
from jax import lax
from jax.experimental import pallas as pl
from jax.experimental.pallas import tpu as pltpu
from jax.experimental.pallas import tpu_sc as plsc

NC = 2
NS = 16
NW = NC * NS

BATCH = 4096
SEQ = 200
EMB = 64
TOTAL = BATCH * SEQ
PER_W = TOTAL // NW
CHUNK = 128
CHUNKS = PER_W // CHUNK


def _make_gather():
  mesh = plsc.VectorSubcoreMesh(
      core_axis_name="c", subcore_axis_name="s",
      num_cores=NC, num_subcores=NS)

  @functools.partial(
      pl.kernel,
      out_type=jax.ShapeDtypeStruct((TOTAL, EMB), jnp.float32),
      mesh=mesh,
      scratch_types=[
          pltpu.VMEM((CHUNKS, CHUNK), jnp.int32),
          pltpu.VMEM((CHUNK, EMB), jnp.float32),
          pltpu.SemaphoreType.DMA,
      ],
      compiler_params=pltpu.CompilerParams(use_tc_tiling_on_sc=False),
  )
  def gather_kernel(word_hbm, table_hbm, out_hbm, idx_v, rows_v, sem):
    wid = lax.axis_index("s") * NC + lax.axis_index("c")
    pltpu.sync_copy(word_hbm.at[wid], idx_v)

    def chunk_body(j, carry):
      pltpu.async_copy(table_hbm.at[idx_v.at[j]], rows_v, sem).wait()
      base = (wid * CHUNKS + j) * CHUNK
      pltpu.sync_copy(rows_v, out_hbm.at[pl.ds(base, CHUNK)])
      return carry

    lax.fori_loop(0, CHUNKS, chunk_body, 0)

  return gather_kernel


_gather = _make_gather()


def kernel(WORD, word_table):
  idx = WORD.reshape(NW, CHUNKS, CHUNK)
  out = _gather(idx, word_table)
  return out.reshape(TOTAL, EMB, 1)


# speedup vs baseline: 1.1196x; 1.1196x over previous
import functools

import jax
import jax.numpy as jnp
from jax import lax
from jax.experimental import pallas as pl
from jax.experimental.pallas import tpu as pltpu
from jax.experimental.pallas import tpu_sc as plsc

NC = 2
NS = 16
NW = NC * NS

BATCH = 4096
SEQ = 200
EMB = 64
TOTAL = BATCH * SEQ
PER_W = TOTAL // NW
CHUNK = 128
CHUNKS = PER_W // CHUNK
NBUF = 4


def _make_gather():
  mesh = plsc.VectorSubcoreMesh(
      core_axis_name="c", subcore_axis_name="s",
      num_cores=NC, num_subcores=NS)

  @functools.partial(
      pl.kernel,
      out_type=jax.ShapeDtypeStruct((TOTAL, EMB), jnp.float32),
      mesh=mesh,
      scratch_types=[
          pltpu.VMEM((CHUNKS, CHUNK), jnp.int32),
          [pltpu.VMEM((CHUNK, EMB), jnp.float32)] * NBUF,
          [pltpu.SemaphoreType.DMA] * NBUF,
      ],
      compiler_params=pltpu.CompilerParams(use_tc_tiling_on_sc=False),
  )
  def gather_kernel(word_hbm, table_hbm, out_hbm, idx_v, bufs, sems):
    wid = lax.axis_index("s") * NC + lax.axis_index("c")
    pltpu.sync_copy(word_hbm.at[wid], idx_v)

    for b in range(NBUF):
      pltpu.async_copy(table_hbm.at[idx_v.at[b]], bufs[b], sems[b])

    def outer(j0, carry):
      for b in range(NBUF):
        j = j0 + b
        pltpu.make_async_copy(table_hbm.at[idx_v.at[j]], bufs[b],
                              sems[b]).wait()
        base = (wid * CHUNKS + j) * CHUNK
        pltpu.sync_copy(bufs[b], out_hbm.at[pl.ds(base, CHUNK)])

        @pl.when(j + NBUF < CHUNKS)
        def _():
          pltpu.async_copy(table_hbm.at[idx_v.at[j + NBUF]], bufs[b],
                           sems[b])
      return carry

    lax.fori_loop(0, CHUNKS // NBUF, lambda i, c: outer(i * NBUF, c), 0)

  return gather_kernel


_gather = _make_gather()


def kernel(WORD, word_table):
  idx = WORD.reshape(NW, CHUNKS, CHUNK)
  out = _gather(idx, word_table)
  return out.reshape(TOTAL, EMB, 1)
